# baseline (speedup 1.0000x reference)
import functools

import jax
import jax.numpy as jnp
from jax.experimental import pallas as pl
from jax.experimental.pallas import tpu as pltpu

N = 16384
V = 100000
VP = 100352
D = 32
K = 8
H = 64

QB = 16
VT = 128
NT = VP // VT
NP = 16
TP = NT // NP
NB = 3
CW = NP * NB * VT

BIG = 3.0e38
PAD_COORD = 1.0e4
IBIG = 2 ** 30


def _knn_body(x_ref, vt_ref, idx_ref, cd_ref, ci_ref):
    x0 = x_ref[:, 0:1]
    x1 = x_ref[:, 1:2]
    x2 = x_ref[:, 2:3]
    xsq = x_ref[:, 3:4]
    lane = jax.lax.broadcasted_iota(jnp.int32, (QB, VT), 1)

    for p in range(NP):
        m1 = jnp.full((QB, VT), BIG, jnp.float32)
        m2 = m1
        m3 = m1
        i1 = jnp.zeros((QB, VT), jnp.int32)
        i2 = i1
        i3 = i1

        def tile(t, carry, p=p):
            m1, m2, m3, i1, i2, i3 = carry
            off = pl.multiple_of(p * (TP * VT) + t * VT, VT)
            v = vt_ref[:, pl.ds(off, VT)]
            dot = (x0 * v[0:1, :] + x1 * v[1:2, :]) + x2 * v[2:3, :]
            d = (xsq - 2.0 * dot) + v[3:4, :]
            ii = lane + off
            c1 = d < m1
            c2 = d < m2
            c3 = d < m3
            m3n = jnp.where(c2, m2, jnp.where(c3, d, m3))
            i3n = jnp.where(c2, i2, jnp.where(c3, ii, i3))
            m2n = jnp.where(c1, m1, jnp.where(c2, d, m2))
            i2n = jnp.where(c1, i1, jnp.where(c2, ii, i2))
            m1n = jnp.minimum(d, m1)
            i1n = jnp.where(c1, ii, i1)
            return m1n, m2n, m3n, i1n, i2n, i3n

        m1, m2, m3, i1, i2, i3 = jax.lax.fori_loop(
            0, TP, tile, (m1, m2, m3, i1, i2, i3))
        base = p * (NB * VT)
        cd_ref[:, base:base + VT] = m1
        cd_ref[:, base + VT:base + 2 * VT] = m2
        cd_ref[:, base + 2 * VT:base + 3 * VT] = m3
        ci_ref[:, base:base + VT] = i1
        ci_ref[:, base + VT:base + 2 * VT] = i2
        ci_ref[:, base + 2 * VT:base + 3 * VT] = i3

    ci = ci_ref[:, :]
    for k in range(K):
        d = cd_ref[:, :]
        m = jnp.min(d, axis=1, keepdims=True)
        eq = d == m
        am = jnp.min(jnp.where(eq, ci, IBIG), axis=1, keepdims=True)
        idx_ref[:, k] = am[:, 0]
        cd_ref[:, :] = jnp.where(eq & (ci == am), BIG, d)


def _round_bf16(x):
    u = jax.lax.bitcast_convert_type(x, jnp.uint32)
    u = (u + jnp.uint32(0x7FFF) + ((u >> 16) & jnp.uint32(1))) & jnp.uint32(0xFFFF0000)
    return jax.lax.bitcast_convert_type(u, jnp.float32)


def _sqnorm(a):
    return (a[:, 0] * a[:, 0] + a[:, 2] * a[:, 2]) + a[:, 1] * a[:, 1]


@functools.partial(jax.jit, static_argnums=())
def _knn(xyz, mesh_vertices):
    vpad = jnp.pad(mesh_vertices, ((0, VP - V), (0, 0)),
                   constant_values=PAD_COORD)
    vt = jnp.concatenate([_round_bf16(vpad), _sqnorm(vpad)[:, None]],
                         axis=1).T
    xq = jnp.concatenate([_round_bf16(xyz), _sqnorm(xyz)[:, None]],
                         axis=1)
    idx = pl.pallas_call(
        _knn_body,
        grid=(N // QB,),
        in_specs=[
            pl.BlockSpec((QB, 4), lambda i: (i, 0)),
            pl.BlockSpec((4, VP), lambda i: (0, 0)),
        ],
        out_specs=pl.BlockSpec((QB, K), lambda i: (i, 0)),
        out_shape=jax.ShapeDtypeStruct((N, K), jnp.int32),
        scratch_shapes=[pltpu.VMEM((QB, CW), jnp.float32),
                        pltpu.VMEM((QB, CW), jnp.int32)],
    )(xq, vt)
    return idx


def kernel(xyz, view_dirs, mesh_vertices, color_features, edit_color_features,
           geo_features, main_mask, W1, b1, W2, b2, Ws1, bs1, Ws2, bs2, Wg, bg):
    idx = _knn(xyz, mesh_vertices)

    neigh = mesh_vertices[idx]
    diff = xyz[:, None, :] - neigh
    ds = jnp.linalg.norm(diff, axis=-1)
    w = 1.0 / (ds + 1e-8)
    w = w / jnp.sum(w, axis=-1, keepdims=True)
    nabla = jnp.sum(w[..., None] * diff, axis=-2)
    nabla = nabla / (jnp.linalg.norm(nabla, axis=-1, keepdims=True) + 1e-8)
    feat = jnp.sum(w[..., None] * color_features[idx], axis=-2)
    geo = jnp.sum(w[..., None] * geo_features[idx], axis=-2)
    sdf = (geo @ Wg + bg).squeeze(-1)
    h = jax.nn.relu(jnp.concatenate([feat, view_dirs, nabla], axis=-1) @ W1 + b1)
    colors = jax.nn.sigmoid(h @ W2 + b2)
    mg = main_mask[idx]
    paint_region = jnp.sum(mg.astype(jnp.int32), axis=-1) >= K
    sw = w * mg.astype(w.dtype)
    sw = sw / (jnp.sum(sw, axis=-1, keepdims=True) + 1e-8)
    sfeat = jnp.sum(sw[..., None] * edit_color_features[idx], axis=-2)
    hs = jax.nn.relu(jnp.concatenate([sfeat, view_dirs, nabla], axis=-1) @ Ws1 + bs1)
    slave_color = jax.nn.sigmoid(hs @ Ws2 + bs2)
    blend_color = jnp.where(paint_region[:, None], slave_color, colors)
    return sdf, blend_color


# speedup vs baseline: 2.9498x; 2.9498x over previous
import functools

import jax
import jax.numpy as jnp
from jax.experimental import pallas as pl
from jax.experimental.pallas import tpu as pltpu

N = 16384
V = 100000
VP = 100352
D = 32
K = 8
H = 64

QB = 16
VT = 512
NT = VP // VT
NP = 4
TP = NT // NP
NB = 3
CW = NP * NB * VT

BIG = 3.0e38
PAD_COORD = 1.0e4
IBIG = 2 ** 30


def _knn_body(x_ref, vt_ref, idx_ref, cd_ref, ci_ref):
    x0 = x_ref[:, 0:1]
    x1 = x_ref[:, 1:2]
    x2 = x_ref[:, 2:3]
    xsq = x_ref[:, 3:4]
    lane = jax.lax.broadcasted_iota(jnp.int32, (QB, VT), 1)

    for p in range(NP):
        m1 = jnp.full((QB, VT), BIG, jnp.float32)
        m2 = m1
        m3 = m1
        i1 = jnp.zeros((QB, VT), jnp.int32)
        i2 = i1
        i3 = i1

        def tile(t, carry, p=p):
            m1, m2, m3, i1, i2, i3 = carry
            off = pl.multiple_of(p * (TP * VT) + t * VT, VT)
            v = vt_ref[:, pl.ds(off, VT)]
            dot = (x0 * v[0:1, :] + x1 * v[1:2, :]) + x2 * v[2:3, :]
            d = (xsq - 2.0 * dot) + v[3:4, :]
            ii = lane + off
            c1 = d < m1
            c2 = d < m2
            c3 = d < m3
            m3n = jnp.where(c2, m2, jnp.where(c3, d, m3))
            i3n = jnp.where(c2, i2, jnp.where(c3, ii, i3))
            m2n = jnp.where(c1, m1, jnp.where(c2, d, m2))
            i2n = jnp.where(c1, i1, jnp.where(c2, ii, i2))
            m1n = jnp.minimum(d, m1)
            i1n = jnp.where(c1, ii, i1)
            return m1n, m2n, m3n, i1n, i2n, i3n

        m1, m2, m3, i1, i2, i3 = jax.lax.fori_loop(
            0, TP, tile, (m1, m2, m3, i1, i2, i3))
        base = p * (NB * VT)
        cd_ref[:, base:base + VT] = m1
        cd_ref[:, base + VT:base + 2 * VT] = m2
        cd_ref[:, base + 2 * VT:base + 3 * VT] = m3
        ci_ref[:, base:base + VT] = i1
        ci_ref[:, base + VT:base + 2 * VT] = i2
        ci_ref[:, base + 2 * VT:base + 3 * VT] = i3

    ci = ci_ref[:, :]
    for k in range(K):
        d = cd_ref[:, :]
        m = jnp.min(d, axis=1, keepdims=True)
        eq = d == m
        am = jnp.min(jnp.where(eq, ci, IBIG), axis=1, keepdims=True)
        idx_ref[:, k] = am[:, 0]
        cd_ref[:, :] = jnp.where(eq & (ci == am), BIG, d)


def _round_bf16(x):
    u = jax.lax.bitcast_convert_type(x, jnp.uint32)
    u = (u + jnp.uint32(0x7FFF) + ((u >> 16) & jnp.uint32(1))) & jnp.uint32(0xFFFF0000)
    return jax.lax.bitcast_convert_type(u, jnp.float32)


def _sqnorm(a):
    return (a[:, 0] * a[:, 0] + a[:, 2] * a[:, 2]) + a[:, 1] * a[:, 1]


@functools.partial(jax.jit, static_argnums=())
def _knn(xyz, mesh_vertices):
    vpad = jnp.pad(mesh_vertices, ((0, VP - V), (0, 0)),
                   constant_values=PAD_COORD)
    vt = jnp.concatenate([_round_bf16(vpad), _sqnorm(vpad)[:, None]],
                         axis=1).T
    xq = jnp.concatenate([_round_bf16(xyz), _sqnorm(xyz)[:, None]],
                         axis=1)
    idx = pl.pallas_call(
        _knn_body,
        grid=(N // QB,),
        in_specs=[
            pl.BlockSpec((QB, 4), lambda i: (i, 0)),
            pl.BlockSpec((4, VP), lambda i: (0, 0)),
        ],
        out_specs=pl.BlockSpec((QB, K), lambda i: (i, 0)),
        out_shape=jax.ShapeDtypeStruct((N, K), jnp.int32),
        scratch_shapes=[pltpu.VMEM((QB, CW), jnp.float32),
                        pltpu.VMEM((QB, CW), jnp.int32)],
    )(xq, vt)
    return idx


def kernel(xyz, view_dirs, mesh_vertices, color_features, edit_color_features,
           geo_features, main_mask, W1, b1, W2, b2, Ws1, bs1, Ws2, bs2, Wg, bg):
    idx = _knn(xyz, mesh_vertices)

    neigh = mesh_vertices[idx]
    diff = xyz[:, None, :] - neigh
    ds = jnp.linalg.norm(diff, axis=-1)
    w = 1.0 / (ds + 1e-8)
    w = w / jnp.sum(w, axis=-1, keepdims=True)
    nabla = jnp.sum(w[..., None] * diff, axis=-2)
    nabla = nabla / (jnp.linalg.norm(nabla, axis=-1, keepdims=True) + 1e-8)
    feat = jnp.sum(w[..., None] * color_features[idx], axis=-2)
    geo = jnp.sum(w[..., None] * geo_features[idx], axis=-2)
    sdf = (geo @ Wg + bg).squeeze(-1)
    h = jax.nn.relu(jnp.concatenate([feat, view_dirs, nabla], axis=-1) @ W1 + b1)
    colors = jax.nn.sigmoid(h @ W2 + b2)
    mg = main_mask[idx]
    paint_region = jnp.sum(mg.astype(jnp.int32), axis=-1) >= K
    sw = w * mg.astype(w.dtype)
    sw = sw / (jnp.sum(sw, axis=-1, keepdims=True) + 1e-8)
    sfeat = jnp.sum(sw[..., None] * edit_color_features[idx], axis=-2)
    hs = jax.nn.relu(jnp.concatenate([sfeat, view_dirs, nabla], axis=-1) @ Ws1 + bs1)
    slave_color = jax.nn.sigmoid(hs @ Ws2 + bs2)
    blend_color = jnp.where(paint_region[:, None], slave_color, colors)
    return sdf, blend_color
